# Initial kernel scaffold; baseline (speedup 1.0000x reference)
import functools

import jax
import jax.numpy as jnp
from jax import lax
from jax.experimental import pallas as pl
from jax.experimental.pallas import tpu as pltpu
from jax.experimental.pallas import tpu_sc as plsc

N = 10000
E = 320000
D = 128
G = 64
NW = 32
EPW = E // NW
CH = 80
NCH = EPW // CH
RPT = N // 16
BLK = 2000
NBLK = N // BLK

_mesh = plsc.VectorSubcoreMesh(core_axis_name="c", subcore_axis_name="s")


def _k1_body(x_ref, w1_ref, b_ref, y_ref, cnt_ref):
    i = pl.program_id(0)
    y_ref[...] = lax.dot_general(
        x_ref[...], w1_ref[...], (((1,), (1,)), ((), ())),
        preferred_element_type=jnp.float32)
    oh = (b_ref[...] == lax.broadcasted_iota(jnp.int32, (1, G), 1)
          ).astype(jnp.float32)
    part = jnp.sum(oh, axis=0, keepdims=True)

    @pl.when(i == 0)
    def _():
        cnt_ref[...] = jnp.zeros_like(cnt_ref)

    cnt_ref[...] += part


def _k1(x, w1, batch2d):
    return pl.pallas_call(
        _k1_body,
        grid=(NBLK,),
        in_specs=[
            pl.BlockSpec((BLK, D), lambda i: (i, 0)),
            pl.BlockSpec((D, D), lambda i: (0, 0)),
            pl.BlockSpec((BLK, 1), lambda i: (i, 0)),
        ],
        out_specs=[
            pl.BlockSpec((BLK, D), lambda i: (i, 0)),
            pl.BlockSpec((1, G), lambda i: (0, 0)),
        ],
        out_shape=[
            jax.ShapeDtypeStruct((N, D), jnp.float32),
            jax.ShapeDtypeStruct((1, G), jnp.float32),
        ],
    )(x, w1, batch2d)


@functools.partial(
    pl.kernel,
    out_type=jax.ShapeDtypeStruct((2, N, D), jnp.float32),
    mesh=_mesh,
    scratch_types=[
        pltpu.VMEM((NCH, CH), jnp.int32),
        pltpu.VMEM((NCH, CH), jnp.int32),
        pltpu.VMEM((EPW,), jnp.float32),
        pltpu.VMEM((CH, D), jnp.float32),
        pltpu.VMEM((125, D), jnp.float32),
        pltpu.VMEM_SHARED((N, D), jnp.float32),
        pltpu.SemaphoreType.DMA,
    ],
)
def _k2(y_hbm, src_hbm, dst_hbm, ew_hbm, out_hbm,
        srcb, dstb, ewb, rows, zb, agg_sh, sem):
    c = lax.axis_index("c")
    s = lax.axis_index("s")
    w = s * 2 + c
    pltpu.sync_copy(src_hbm.at[w], srcb)
    pltpu.sync_copy(dst_hbm.at[w], dstb)
    pltpu.sync_copy(ew_hbm.at[w], ewb)

    z16 = jnp.zeros((16,), jnp.float32)

    def zrow(i, carry):
        for f in range(D // 16):
            zb[i, pl.ds(f * 16, 16)] = z16
        return carry

    lax.fori_loop(0, 125, zrow, 0)
    for j in range(RPT // 125):
        pltpu.sync_copy(zb, agg_sh.at[pl.ds(s * RPT + j * 125, 125)])
    plsc.subcore_barrier()

    def chunk(i, carry):
        pltpu.async_copy(y_hbm.at[srcb.at[i]], rows, sem).wait()

        def scale(e, carry2):
            wv = ewb[i * CH + e]
            for f in range(D // 16):
                rows[e, pl.ds(f * 16, 16)] = rows[e, pl.ds(f * 16, 16)] * wv
            return carry2

        lax.fori_loop(0, CH, scale, 0)
        pltpu.sync_copy(rows, agg_sh.at[dstb.at[i]], add=True)
        return carry

    lax.fori_loop(0, NCH, chunk, 0)
    plsc.subcore_barrier()
    for j in range(RPT // 125):
        pltpu.sync_copy(agg_sh.at[pl.ds(s * RPT + j * 125, 125)],
                        out_hbm.at[c, pl.ds(s * RPT + j * 125, 125)])


def _k3_body(a0_ref, a1_ref, cnt_ref, b_ref, b1_ref, w2_ref, s_ref):
    a = a0_ref[...] + a1_ref[...]
    oh = (b_ref[...] == lax.broadcasted_iota(jnp.int32, (1, G), 1)
          ).astype(jnp.float32)
    n_each = lax.dot_general(oh, cnt_ref[...], (((1,), (1,)), ((), ())),
                             preferred_element_type=jnp.float32)
    ninv = 1.0 / jnp.maximum(n_each, 1.0)
    h = jax.nn.sigmoid(a * ninv + b1_ref[...])
    s_ref[...] = lax.dot_general(h, w2_ref[...], (((1,), (1,)), ((), ())),
                                 preferred_element_type=jnp.float32)


def _k3(a0, a1, cnt, batch2d, b1r, w2):
    return pl.pallas_call(
        _k3_body,
        grid=(NBLK,),
        in_specs=[
            pl.BlockSpec((BLK, D), lambda i: (i, 0)),
            pl.BlockSpec((BLK, D), lambda i: (i, 0)),
            pl.BlockSpec((1, G), lambda i: (0, 0)),
            pl.BlockSpec((BLK, 1), lambda i: (i, 0)),
            pl.BlockSpec((1, D), lambda i: (0, 0)),
            pl.BlockSpec((1, D), lambda i: (0, 0)),
        ],
        out_specs=pl.BlockSpec((BLK, 1), lambda i: (i, 0)),
        out_shape=jax.ShapeDtypeStruct((N, 1), jnp.float32),
    )(a0, a1, cnt, batch2d, b1r, w2)


@functools.partial(
    pl.kernel,
    out_type=jax.ShapeDtypeStruct((2, N), jnp.float32),
    mesh=_mesh,
    scratch_types=[
        pltpu.VMEM((N,), jnp.float32),
        pltpu.VMEM((EPW,), jnp.int32),
        pltpu.VMEM((NCH, CH), jnp.int32),
        pltpu.VMEM((EPW,), jnp.float32),
        pltpu.VMEM((CH,), jnp.float32),
        pltpu.VMEM((N,), jnp.float32),
        pltpu.VMEM_SHARED((N,), jnp.float32),
    ],
)
def _k4(s_hbm, srcf_hbm, dst_hbm, ew_hbm, out_hbm,
        sb, srcb, dstb, ewb, vb, zb, t_sh):
    c = lax.axis_index("c")
    s = lax.axis_index("s")
    w = s * 2 + c
    pltpu.sync_copy(s_hbm, sb)
    pltpu.sync_copy(srcf_hbm.at[w], srcb)
    pltpu.sync_copy(dst_hbm.at[w], dstb)
    pltpu.sync_copy(ew_hbm.at[w], ewb)

    @pl.when(s == 0)
    def _():
        z16 = jnp.zeros((16,), jnp.float32)

        def zr(i, carry):
            zb[pl.ds(i * 16, 16)] = z16
            return carry

        lax.fori_loop(0, N // 16, zr, 0)
        pltpu.sync_copy(zb, t_sh)

    plsc.subcore_barrier()

    def chunk(i, carry):
        for g in range(CH // 16):
            idx16 = srcb[pl.ds(i * CH + g * 16, 16)]
            sv = plsc.load_gather(sb, [idx16])
            wv = ewb[pl.ds(i * CH + g * 16, 16)]
            vb[pl.ds(g * 16, 16)] = sv * wv
        pltpu.sync_copy(vb, t_sh.at[dstb.at[i]], add=True)
        return carry

    lax.fori_loop(0, NCH, chunk, 0)
    plsc.subcore_barrier()

    @pl.when(s == 0)
    def _():
        pltpu.sync_copy(t_sh, out_hbm.at[c])


def _k5_body(t0_ref, t1_ref, b_ref, cnt_ref, b2_ref, o_ref):
    i = pl.program_id(0)
    t = t0_ref[...] + t1_ref[...]
    oh = (b_ref[...] == lax.broadcasted_iota(jnp.int32, (1, G), 1)
          ).astype(jnp.float32)
    part = lax.dot_general(t, oh, (((1,), (0,)), ((), ())),
                           preferred_element_type=jnp.float32)

    @pl.when(i == 0)
    def _():
        o_ref[...] = jnp.zeros_like(o_ref)

    o_ref[...] += part

    @pl.when(i == NBLK - 1)
    def _():
        cnt = cnt_ref[...]
        o_ref[...] = o_ref[...] / jnp.maximum(cnt * cnt, 1.0) + b2_ref[...]


def _k5(t0, t1, batch2d, cnt, b2r):
    return pl.pallas_call(
        _k5_body,
        grid=(NBLK,),
        in_specs=[
            pl.BlockSpec((1, BLK), lambda i: (0, i)),
            pl.BlockSpec((1, BLK), lambda i: (0, i)),
            pl.BlockSpec((BLK, 1), lambda i: (i, 0)),
            pl.BlockSpec((1, G), lambda i: (0, 0)),
            pl.BlockSpec((1, 1), lambda i: (0, 0)),
        ],
        out_specs=pl.BlockSpec((1, G), lambda i: (0, 0)),
        out_shape=jax.ShapeDtypeStruct((1, G), jnp.float32),
    )(t0, t1, batch2d, cnt, b2r)


def kernel(x, edge_index, edge_weight, batch, W1, b1, W2, b2):
    src = edge_index[0].astype(jnp.int32)
    dst = edge_index[1].astype(jnp.int32)
    batch2d = batch.astype(jnp.int32).reshape(N, 1)
    src_r = src.reshape(NW, NCH, CH)
    dst_r = dst.reshape(NW, NCH, CH)
    srcf = src.reshape(NW, EPW)
    ew_r = edge_weight.reshape(NW, EPW)
    b1r = b1.reshape(1, D)
    b2r = b2.reshape(1, 1)

    y, cnt = _k1(x, W1, batch2d)
    agg2 = _k2(y, src_r, dst_r, ew_r)
    s = _k3(agg2[0], agg2[1], cnt, batch2d, b1r, W2)
    t2 = _k4(s.reshape(N), srcf, dst_r, ew_r)
    p = _k5(t2[0].reshape(1, N), t2[1].reshape(1, N), batch2d, cnt, b2r)
    return p.reshape(G, 1)


# speedup vs baseline: 8.4120x; 8.4120x over previous
import functools

import jax
import jax.numpy as jnp
from jax import lax
from jax.experimental import pallas as pl
from jax.experimental.pallas import tpu as pltpu
from jax.experimental.pallas import tpu_sc as plsc

N = 10000
E = 320000
D = 128
G = 64
NW = 32
EPW = E // NW
CH = 80
NCH = EPW // CH
RPT = N // 16
BLK = 2000
NBLK = N // BLK

def _sc_mesh():
    return plsc.VectorSubcoreMesh(core_axis_name="c", subcore_axis_name="s",
                                  num_cores=2, num_subcores=16)


def _k1_body(x_ref, w1_ref, b_ref, y_ref, cnt_ref):
    i = pl.program_id(0)
    y_ref[...] = lax.dot_general(
        x_ref[...], w1_ref[...], (((1,), (1,)), ((), ())),
        preferred_element_type=jnp.float32)
    oh = (b_ref[...] == lax.broadcasted_iota(jnp.int32, (1, G), 1)
          ).astype(jnp.float32)
    part = jnp.sum(oh, axis=0, keepdims=True)

    @pl.when(i == 0)
    def _():
        cnt_ref[...] = jnp.zeros_like(cnt_ref)

    cnt_ref[...] += part


def _k1(x, w1, batch2d):
    return pl.pallas_call(
        _k1_body,
        grid=(NBLK,),
        in_specs=[
            pl.BlockSpec((BLK, D), lambda i: (i, 0)),
            pl.BlockSpec((D, D), lambda i: (0, 0)),
            pl.BlockSpec((BLK, 1), lambda i: (i, 0)),
        ],
        out_specs=[
            pl.BlockSpec((BLK, D), lambda i: (i, 0)),
            pl.BlockSpec((1, G), lambda i: (0, 0)),
        ],
        out_shape=[
            jax.ShapeDtypeStruct((N, D), jnp.float32),
            jax.ShapeDtypeStruct((1, G), jnp.float32),
        ],
    )(x, w1, batch2d)


@functools.cache
def _k2_kernel():
    return pl.kernel(
        _k2_body,
        out_type=jax.ShapeDtypeStruct((2, N, D), jnp.float32),
        mesh=_sc_mesh(),
        compiler_params=pltpu.CompilerParams(use_tc_tiling_on_sc=False, needs_layout_passes=False),
        scratch_types=[
            pltpu.VMEM((NCH, CH), jnp.int32),
            pltpu.VMEM((NCH, CH), jnp.int32),
            pltpu.VMEM((EPW,), jnp.float32),
            pltpu.VMEM((CH, D), jnp.float32),
            pltpu.VMEM((25, D), jnp.float32),
            pltpu.VMEM_SHARED((N, D), jnp.float32),
            pltpu.SemaphoreType.DMA,
        ],
    )


def _k2(*args):
    return _k2_kernel()(*args)


def _k2_body(y_hbm, src_hbm, dst_hbm, ew_hbm, out_hbm,
        srcb, dstb, ewb, rows, zb, agg_sh, sem):
    c = lax.axis_index("c")
    s = lax.axis_index("s")
    w = s * 2 + c
    pltpu.sync_copy(src_hbm.at[w], srcb)
    pltpu.sync_copy(dst_hbm.at[w], dstb)
    pltpu.sync_copy(ew_hbm.at[w], ewb)

    z16 = jnp.zeros((16,), jnp.float32)

    def zrow(i, carry):
        for f in range(D // 16):
            zb[i, pl.ds(f * 16, 16)] = z16
        return carry

    lax.fori_loop(0, 25, zrow, 0)
    for j in range(RPT // 25):
        pltpu.sync_copy(zb, agg_sh.at[pl.ds(s * RPT + j * 25, 25)])
    plsc.subcore_barrier()

    def chunk(i, carry):
        pltpu.async_copy(y_hbm.at[srcb.at[i]], rows, sem).wait()

        def scale(e, carry2):
            idx = lax.broadcast(i * CH + e, (16,))
            wv = plsc.load_gather(ewb, [idx])
            for f in range(D // 16):
                rows[e, pl.ds(f * 16, 16)] = rows[e, pl.ds(f * 16, 16)] * wv
            return carry2

        lax.fori_loop(0, CH, scale, 0)
        pltpu.sync_copy(rows, agg_sh.at[dstb.at[i]], add=True)
        return carry

    lax.fori_loop(0, NCH, chunk, 0)
    plsc.subcore_barrier()
    pltpu.sync_copy(agg_sh.at[pl.ds(s * RPT, RPT)],
                    out_hbm.at[c, pl.ds(s * RPT, RPT)])


def _k3_body(a0_ref, a1_ref, cnt_ref, b_ref, b1_ref, w2_ref, s_ref):
    a = a0_ref[...] + a1_ref[...]
    oh = (b_ref[...] == lax.broadcasted_iota(jnp.int32, (1, G), 1)
          ).astype(jnp.float32)
    n_each = lax.dot_general(oh, cnt_ref[...], (((1,), (1,)), ((), ())),
                             preferred_element_type=jnp.float32)
    ninv = 1.0 / jnp.maximum(n_each, 1.0)
    h = jax.nn.sigmoid(a * ninv + b1_ref[...])
    s_ref[...] = lax.dot_general(h, w2_ref[...], (((1,), (1,)), ((), ())),
                                 preferred_element_type=jnp.float32)


def _k3(a0, a1, cnt, batch2d, b1r, w2):
    return pl.pallas_call(
        _k3_body,
        grid=(NBLK,),
        in_specs=[
            pl.BlockSpec((BLK, D), lambda i: (i, 0)),
            pl.BlockSpec((BLK, D), lambda i: (i, 0)),
            pl.BlockSpec((1, G), lambda i: (0, 0)),
            pl.BlockSpec((BLK, 1), lambda i: (i, 0)),
            pl.BlockSpec((1, D), lambda i: (0, 0)),
            pl.BlockSpec((1, D), lambda i: (0, 0)),
        ],
        out_specs=pl.BlockSpec((BLK, 1), lambda i: (i, 0)),
        out_shape=jax.ShapeDtypeStruct((N, 1), jnp.float32),
    )(a0, a1, cnt, batch2d, b1r, w2)


@functools.cache
def _k4_kernel():
    return pl.kernel(
        _k4_body,
        out_type=jax.ShapeDtypeStruct((2, N), jnp.float32),
        mesh=_sc_mesh(),
        compiler_params=pltpu.CompilerParams(use_tc_tiling_on_sc=False, needs_layout_passes=False),
        scratch_types=[
            pltpu.VMEM((N,), jnp.float32),
            pltpu.VMEM((EPW,), jnp.int32),
            pltpu.VMEM((NCH, CH), jnp.int32),
            pltpu.VMEM((EPW,), jnp.float32),
            pltpu.VMEM((CH,), jnp.float32),
            pltpu.VMEM((N,), jnp.float32),
            pltpu.VMEM_SHARED((N,), jnp.float32),
        ],
    )


def _k4(*args):
    return _k4_kernel()(*args)


def _k4_body(s_hbm, srcf_hbm, dst_hbm, ew_hbm, out_hbm,
        sb, srcb, dstb, ewb, vb, zb, t_sh):
    c = lax.axis_index("c")
    s = lax.axis_index("s")
    w = s * 2 + c
    pltpu.sync_copy(s_hbm, sb)
    pltpu.sync_copy(srcf_hbm.at[w], srcb)
    pltpu.sync_copy(dst_hbm.at[w], dstb)
    pltpu.sync_copy(ew_hbm.at[w], ewb)

    @pl.when(s == 0)
    def _():
        z16 = jnp.zeros((16,), jnp.float32)

        def zr(i, carry):
            zb[pl.ds(i * 16, 16)] = z16
            return carry

        lax.fori_loop(0, N // 16, zr, 0)
        pltpu.sync_copy(zb, t_sh)

    plsc.subcore_barrier()

    def chunk(i, carry):
        for g in range(CH // 16):
            idx16 = srcb[pl.ds(i * CH + g * 16, 16)]
            sv = plsc.load_gather(sb, [idx16])
            wv = ewb[pl.ds(i * CH + g * 16, 16)]
            vb[pl.ds(g * 16, 16)] = sv * wv
        pltpu.sync_copy(vb, t_sh.at[dstb.at[i]], add=True)
        return carry

    lax.fori_loop(0, NCH, chunk, 0)
    plsc.subcore_barrier()

    @pl.when(s == 0)
    def _():
        pltpu.sync_copy(t_sh, out_hbm.at[c])


def _k5_body(t0_ref, t1_ref, b_ref, cnt_ref, b2_ref, o_ref):
    t = t0_ref[...] + t1_ref[...]
    oh = (b_ref[...] == lax.broadcasted_iota(jnp.int32, (1, G), 1)
          ).astype(jnp.float32)
    acc = lax.dot_general(t, oh, (((1,), (0,)), ((), ())),
                          preferred_element_type=jnp.float32)
    cnt = cnt_ref[...]
    o_ref[...] = acc / jnp.maximum(cnt * cnt, 1.0) + b2_ref[...]


def _k5(t0, t1, batch2d, cnt, b2r):
    return pl.pallas_call(
        _k5_body,
        out_shape=jax.ShapeDtypeStruct((1, G), jnp.float32),
    )(t0, t1, batch2d, cnt, b2r)


def kernel(x, edge_index, edge_weight, batch, W1, b1, W2, b2):
    src = edge_index[0].astype(jnp.int32)
    dst = edge_index[1].astype(jnp.int32)
    batch2d = batch.astype(jnp.int32).reshape(N, 1)
    src_r = src.reshape(NW, NCH, CH)
    dst_r = dst.reshape(NW, NCH, CH)
    srcf = src.reshape(NW, EPW)
    ew_r = edge_weight.reshape(NW, EPW)
    b1r = b1.reshape(1, D)
    b2r = b2.reshape(1, 1)

    y, cnt = _k1(x, W1, batch2d)
    agg2 = _k2(y, src_r, dst_r, ew_r)
    s = _k3(agg2[0], agg2[1], cnt, batch2d, b1r, W2)
    t2 = _k4(s.reshape(N), srcf, dst_r, ew_r)
    p = _k5(t2[0].reshape(1, N), t2[1].reshape(1, N), batch2d, cnt, b2r)
    return p.reshape(G, 1)
